# Initial kernel scaffold; baseline (speedup 1.0000x reference)
import functools

import jax
import jax.numpy as jnp
from jax import lax
from jax.experimental import pallas as pl
from jax.experimental.pallas import tpu as pltpu
from jax.experimental.pallas import tpu_sc as plsc

VOCAB = 1000000
EMBED = 64
B = 16384
L = 50
HID = 50
NCLS = 2

NC = 2
NS = 16
NW = NC * NS
ROWS_W = B // NW
RPC = 2
IDXC = RPC * L
CHUNKS = ROWS_W // RPC
NV = EMBED // 16


def _sc_body(text_hbm, table_hbm, out_hbm, idx_v, buf0, buf1, out_v, sem0, sem1):
    wid = lax.axis_index("s") * NC + lax.axis_index("c")
    pltpu.sync_copy(text_hbm.at[wid], idx_v)

    bufs = (buf0, buf1)
    sems = (sem0, sem1)

    def accumulate(buf, g):
        for r in range(RPC):
            row = g * RPC + r
            for k in range(NV):
                acc = buf[r * L, pl.ds(k * 16, 16)]
                for l in range(1, L):
                    acc = acc + buf[r * L + l, pl.ds(k * 16, 16)]
                out_v[row, pl.ds(k * 16, 16)] = acc

    pltpu.async_copy(table_hbm.at[idx_v.at[0]], buf0, sem0)
    pltpu.async_copy(table_hbm.at[idx_v.at[1]], buf1, sem1)

    def body(gp, _):
        for b in range(2):
            g = 2 * gp + b
            pltpu.make_async_copy(table_hbm.at[idx_v.at[g]], bufs[b], sems[b]).wait()
            accumulate(bufs[b], g)
            nxt = g + 2

            @pl.when(nxt < CHUNKS)
            def _():
                pltpu.async_copy(table_hbm.at[idx_v.at[nxt]], bufs[b], sems[b])

        return 0

    lax.fori_loop(0, CHUNKS // 2, body, 0)

    pltpu.sync_copy(out_v, out_hbm.at[pl.ds(wid * ROWS_W, ROWS_W)])


def _sc_pool(text_r, table):
    mesh = plsc.VectorSubcoreMesh(core_axis_name="c", subcore_axis_name="s")
    return pl.kernel(
        _sc_body,
        out_type=jax.ShapeDtypeStruct((B, EMBED), jnp.float32),
        mesh=mesh,
        scratch_types=[
            pltpu.VMEM((CHUNKS, IDXC), jnp.int32),
            pltpu.VMEM((IDXC, EMBED), jnp.float32),
            pltpu.VMEM((IDXC, EMBED), jnp.float32),
            pltpu.VMEM((ROWS_W, EMBED), jnp.float32),
            pltpu.SemaphoreType.DMA,
            pltpu.SemaphoreType.DMA,
        ],
    )(text_r, table)


def _mlp_body(pooled_ref, len_ref, wa_ref, wl_ref, b1_ref, w2_ref, b2_ref, out_ref):
    p = pooled_ref[...]
    h = jnp.dot(p, wa_ref[...], preferred_element_type=jnp.float32)
    h = h + len_ref[...] * wl_ref[...] + b1_ref[...]
    h = jnp.maximum(h, 0.0)
    out_ref[...] = jnp.dot(h, w2_ref[...], preferred_element_type=jnp.float32) + b2_ref[...]


def _mlp(pooled, len_col, wa, wl, b1r, w2t, b2r):
    return pl.pallas_call(
        _mlp_body,
        out_shape=jax.ShapeDtypeStruct((B, NCLS), jnp.float32),
    )(pooled, len_col, wa, wl, b1r, w2t, b2r)


def kernel(text, text_len, table, W1, b1, W2, b2):
    text_r = text.reshape(NW, CHUNKS, IDXC)
    pooled = _sc_pool(text_r, table)

    wa = W1[:, :EMBED].T * (1.0 / L)
    wl = W1[:, EMBED].reshape(1, HID)
    len_col = text_len.astype(jnp.float32).reshape(B, 1)
    out = _mlp(pooled, len_col, wa, wl, b1.reshape(1, HID), W2.T, b2.reshape(1, NCLS))
    return out


# speedup vs baseline: 2.1502x; 2.1502x over previous
import functools

import jax
import jax.numpy as jnp
from jax import lax
from jax.experimental import pallas as pl
from jax.experimental.pallas import tpu as pltpu
from jax.experimental.pallas import tpu_sc as plsc

VOCAB = 1000000
EMBED = 64
B = 16384
L = 50
HID = 50
NCLS = 2

NC = 2
NS = 16
NW = NC * NS
ROWS_W = B // NW
RPC = 2
IDXC = RPC * L
CHUNKS = ROWS_W // RPC
NV = EMBED // 16


def _sc_body(text_hbm, table_hbm, out_hbm, idx_v, buf0, buf1, out_v, sem0, sem1):
    wid = lax.axis_index("s") * NC + lax.axis_index("c")
    pltpu.sync_copy(text_hbm.at[wid], idx_v)

    bufs = (buf0, buf1)
    sems = (sem0, sem1)

    def accumulate(buf, g):
        for r in range(RPC):
            row = g * RPC + r
            for k in range(NV):
                acc = buf[r * L, pl.ds(k * 16, 16)]
                for l in range(1, L):
                    acc = acc + buf[r * L + l, pl.ds(k * 16, 16)]
                out_v[row, pl.ds(k * 16, 16)] = acc

    pltpu.async_copy(table_hbm.at[idx_v.at[0]], buf0, sem0)
    pltpu.async_copy(table_hbm.at[idx_v.at[1]], buf1, sem1)

    def body(gp, _):
        for b in range(2):
            g = 2 * gp + b
            pltpu.make_async_copy(table_hbm.at[idx_v.at[g]], bufs[b], sems[b]).wait()
            accumulate(bufs[b], g)
            nxt = g + 2

            @pl.when(nxt < CHUNKS)
            def _():
                pltpu.async_copy(table_hbm.at[idx_v.at[nxt]], bufs[b], sems[b])

        return 0

    lax.fori_loop(0, CHUNKS // 2, body, 0)

    pltpu.sync_copy(out_v, out_hbm.at[pl.ds(wid * ROWS_W, ROWS_W)])


def _sc_pool(text_r, table):
    mesh = plsc.VectorSubcoreMesh(core_axis_name="c", subcore_axis_name="s")
    return pl.kernel(
        _sc_body,
        out_type=jax.ShapeDtypeStruct((B, EMBED), jnp.float32),
        mesh=mesh,
        scratch_types=[
            pltpu.VMEM((CHUNKS, IDXC), jnp.int32),
            pltpu.VMEM((IDXC, EMBED), jnp.float32),
            pltpu.VMEM((IDXC, EMBED), jnp.float32),
            pltpu.VMEM((ROWS_W, EMBED), jnp.float32),
            pltpu.SemaphoreType.DMA,
            pltpu.SemaphoreType.DMA,
        ],
        compiler_params=pltpu.CompilerParams(use_tc_tiling_on_sc=False),
    )(text_r, table)


def _mlp_body(pooled_ref, len_ref, wa_ref, wl_ref, b1_ref, w2_ref, b2_ref, out_ref):
    p = pooled_ref[...]
    h = jnp.dot(p, wa_ref[...], preferred_element_type=jnp.float32)
    h = h + len_ref[...] * wl_ref[...] + b1_ref[...]
    h = jnp.maximum(h, 0.0)
    out_ref[...] = jnp.dot(h, w2_ref[...], preferred_element_type=jnp.float32) + b2_ref[...]


def _mlp(pooled, len_col, wa, wl, b1r, w2t, b2r):
    return pl.pallas_call(
        _mlp_body,
        out_shape=jax.ShapeDtypeStruct((B, NCLS), jnp.float32),
    )(pooled, len_col, wa, wl, b1r, w2t, b2r)


def kernel(text, text_len, table, W1, b1, W2, b2):
    text_r = text.reshape(NW, CHUNKS, IDXC)
    pooled = _sc_pool(text_r, table)

    wa = W1[:, :EMBED].T * (1.0 / L)
    wl = W1[:, EMBED].reshape(1, HID)
    len_col = text_len.astype(jnp.float32).reshape(B, 1)
    out = _mlp(pooled, len_col, wa, wl, b1.reshape(1, HID), W2.T, b2.reshape(1, NCLS))
    return out
